# Initial kernel scaffold; baseline (speedup 1.0000x reference)
import jax
import jax.numpy as jnp
from jax.experimental import pallas as pl


def kernel(actions, table, W, b):
    raise NotImplementedError("write your pallas kernel here")


# speedup vs baseline: 1.5543x; 1.5543x over previous
import functools

import jax
import jax.numpy as jnp
from jax import lax
from jax.experimental import pallas as pl
from jax.experimental.pallas import tpu as pltpu
from jax.experimental.pallas import tpu_sc as plsc

EMBED = 64
BATCH = 16384

NC = 2
NS = 16
NW = NC * NS
JOBS = 2 * BATCH
PER_W = JOBS // NW
CHUNK = 128
NCHUNK = PER_W // CHUNK


def _gather_body(idx_hbm, table_hbm, ex_hbm, idx_v, rows_v, sem):
    wid = lax.axis_index("s") * NC + lax.axis_index("c")
    pltpu.sync_copy(idx_hbm.at[wid], idx_v)
    copies = []
    for j in range(NCHUNK):
        copies.append(
            pltpu.async_copy(table_hbm.at[idx_v.at[j]], rows_v.at[j], sem))
    for c in copies:
        c.wait()
    pltpu.sync_copy(rows_v, ex_hbm.at[wid])


@functools.partial(
    pl.kernel,
    mesh=plsc.VectorSubcoreMesh(core_axis_name="c", subcore_axis_name="s"),
    out_type=jax.ShapeDtypeStruct((NW, NCHUNK, CHUNK, EMBED), jnp.float32),
    scratch_types=[
        pltpu.VMEM((NCHUNK, CHUNK), jnp.int32),
        pltpu.VMEM((NCHUNK, CHUNK, EMBED), jnp.float32),
        pltpu.SemaphoreType.DMA,
    ],
    compiler_params=pltpu.CompilerParams(use_tc_tiling_on_sc=False),
)
def _gather(idx_hbm, table_hbm, ex_hbm, idx_v, rows_v, sem):
    _gather_body(idx_hbm, table_hbm, ex_hbm, idx_v, rows_v, sem)


BM = 2048


def _mm_body(enc_ref, w_ref, b_ref, out_ref):
    out_ref[...] = (
        jnp.dot(enc_ref[...], w_ref[...], preferred_element_type=jnp.float32)
        + b_ref[...])


def _matmul(encoded, W, b2d):
    return pl.pallas_call(
        _mm_body,
        grid=(BATCH // BM,),
        in_specs=[
            pl.BlockSpec((BM, 2 * EMBED), lambda i: (i, 0)),
            pl.BlockSpec((2 * EMBED, EMBED), lambda i: (0, 0)),
            pl.BlockSpec((1, EMBED), lambda i: (0, 0)),
        ],
        out_specs=pl.BlockSpec((BM, EMBED), lambda i: (i, 0)),
        out_shape=jax.ShapeDtypeStruct((BATCH, EMBED), jnp.float32),
    )(encoded, W, b2d)


def kernel(actions, table, W, b):
    idx = actions.astype(jnp.int32).reshape(NW, NCHUNK, CHUNK)
    ex = _gather(idx, table)
    encoded = ex.reshape(BATCH, 2 * EMBED)
    return _matmul(encoded, W, b.reshape(1, EMBED))
